# Initial kernel scaffold; baseline (speedup 1.0000x reference)
import jax
import jax.numpy as jnp
from jax.experimental import pallas as pl


def kernel(cell_type, dose, time, batch, E_cell_type, E_dose, E_time, E_batch, W, b):
    raise NotImplementedError("write your pallas kernel here")


# speedup vs baseline: 3.2957x; 3.2957x over previous
import jax
import jax.numpy as jnp
from jax import lax
from jax.experimental import pallas as pl
from jax.experimental.pallas import tpu as pltpu
from jax.experimental.pallas import tpu_sc as plsc

DIM_ = 128
B_ = 16384
NC_ = 2
NS_ = 16
NW_ = NC_ * NS_
BPW_ = B_ // NW_
NCH_ = BPW_ // 128


def _sc_gather_body(idx_hbm, t0_hbm, t1_hbm, t2_hbm, t3_hbm, out_hbm,
                    idx_v, rows_v, sem):
    wid = lax.axis_index("s") * NC_ + lax.axis_index("c")
    base = wid * BPW_
    pltpu.sync_copy(idx_hbm.at[wid], idx_v)
    for t, tab in enumerate((t0_hbm, t1_hbm, t2_hbm, t3_hbm)):
        copies = []
        for j in range(NCH_):
            copies.append(
                pltpu.async_copy(tab.at[idx_v.at[t, j]],
                                 rows_v.at[pl.ds(j * 128, 128)], sem))
        for c in copies:
            c.wait()
        pltpu.sync_copy(rows_v, out_hbm.at[t, pl.ds(base, BPW_)])


_gather4 = pl.kernel(
    _sc_gather_body,
    out_type=jax.ShapeDtypeStruct((4, B_, DIM_), jnp.float32),
    mesh=plsc.VectorSubcoreMesh(core_axis_name="c", subcore_axis_name="s"),
    scratch_types=[
        pltpu.VMEM((4, NCH_, 128), jnp.int32),
        pltpu.VMEM((BPW_, DIM_), jnp.float32),
        pltpu.SemaphoreType.DMA,
    ],
)


def _proj_body(x_ref, w_ref, b_ref, o_ref):
    acc = jnp.broadcast_to(b_ref[...], o_ref.shape).astype(jnp.float32)
    for t in range(4):
        acc = acc + jnp.dot(x_ref[t], w_ref[t],
                            preferred_element_type=jnp.float32)
    o_ref[...] = acc


def _proj(x, w4, b2, bm=512):
    return pl.pallas_call(
        _proj_body,
        grid=(B_ // bm,),
        in_specs=[
            pl.BlockSpec((4, bm, DIM_), lambda i: (0, i, 0)),
            pl.BlockSpec((4, DIM_, DIM_), lambda i: (0, 0, 0)),
            pl.BlockSpec((1, DIM_), lambda i: (0, 0)),
        ],
        out_specs=pl.BlockSpec((bm, DIM_), lambda i: (i, 0)),
        out_shape=jax.ShapeDtypeStruct((B_, DIM_), jnp.float32),
    )(x, w4, b2)


def kernel(cell_type, dose, time, batch, E_cell_type, E_dose, E_time,
           E_batch, W, b):
    idx = jnp.stack([cell_type.astype(jnp.int32), dose.astype(jnp.int32),
                     time.astype(jnp.int32), batch.astype(jnp.int32)])
    idx = idx.reshape(4, NW_, NCH_, 128).transpose(1, 0, 2, 3)
    gathered = _gather4(idx, E_cell_type, E_dose, E_time, E_batch)
    w4 = W.reshape(4, DIM_, DIM_)
    return _proj(gathered, w4, b.reshape(1, DIM_))
